# Initial kernel scaffold; baseline (speedup 1.0000x reference)
import functools

import jax
import jax.numpy as jnp
from jax import lax
from jax.experimental import pallas as pl
from jax.experimental.pallas import tpu as pltpu
from jax.experimental.pallas import tpu_sc as plsc

N = 10000
E = 320000
F = 128
H = 3
C = 16

NC = 2
NS = 16
NW = NC * NS

K = 128
EPT = ((E + NW * K - 1) // (NW * K)) * K
EP = EPT * NW
NCHUNK = EPT // K

NACC = 10240
RT = NACC // NS

RB = 1024
RE = 1000


def _pre_body(x_ref, w_ref, b_ref, o_ref):
    o_ref[...] = (
        jnp.dot(x_ref[...], w_ref[...], preferred_element_type=jnp.float32)
        + b_ref[...]
    )


_pre_call = pl.pallas_call(
    _pre_body,
    grid=(NACC // RB,),
    in_specs=[
        pl.BlockSpec((RB, F), lambda i: (i, 0)),
        pl.BlockSpec((F, 8), lambda i: (0, 0)),
        pl.BlockSpec((1, 8), lambda i: (0, 0)),
    ],
    out_specs=pl.BlockSpec((RB, 8), lambda i: (i, 0)),
    out_shape=jax.ShapeDtypeStruct((NACC, 8), jnp.float32),
)


def _sc_body(ypack_hbm, src_hbm, dst_hbm, zeros_hbm, out_hbm,
             src_v, dst_v, rows_v, acc_sh, sem):
    c = lax.axis_index("c")
    s = lax.axis_index("s")
    wid = c * NS + s

    pltpu.sync_copy(zeros_hbm.at[pl.ds(s * RT, RT)], acc_sh.at[pl.ds(s * RT, RT)])
    plsc.subcore_barrier()

    @pl.loop(0, NCHUNK)
    def _(g):
        base = wid * EPT + g * K
        pltpu.sync_copy(src_hbm.at[pl.ds(base, K)], src_v)
        pltpu.sync_copy(dst_hbm.at[pl.ds(base, K)], dst_v)
        pltpu.async_copy(ypack_hbm.at[src_v], rows_v, sem).wait()
        pltpu.sync_copy(rows_v, acc_sh.at[dst_v], add=True)

    plsc.subcore_barrier()
    out_base = c * NACC + s * RT
    pltpu.sync_copy(acc_sh.at[pl.ds(s * RT, RT)], out_hbm.at[pl.ds(out_base, RT)])


_sc_call = functools.partial(
    pl.kernel,
    out_type=jax.ShapeDtypeStruct((NC * NACC, 4), jnp.float32),
    mesh=plsc.VectorSubcoreMesh(core_axis_name="c", subcore_axis_name="s"),
    scratch_types=[
        pltpu.VMEM((K,), jnp.int32),
        pltpu.VMEM((K,), jnp.int32),
        pltpu.VMEM((K, 4), jnp.float32),
        pltpu.VMEM_SHARED((NACC, 4), jnp.float32),
        pltpu.SemaphoreType.DMA,
    ],
)(_sc_body)


def _epi_body(a0_ref, a1_ref, pre_ref, w_ref, b_ref, h_ref, z_ref):
    a0 = a0_ref[...]
    a1 = a1_ref[...]
    ssum = a0[:, 0:3] + a1[:, 0:3]
    cnt = a0[:, 3:4] + a1[:, 3:4]
    mean = ssum / jnp.maximum(cnt, 1.0)
    out = pre_ref[:, 4:7] + mean
    h = jnp.maximum(out, 0.0)
    z = jnp.dot(h, w_ref[...], preferred_element_type=jnp.float32) + b_ref[...]
    h_ref[...] = h
    z_ref[...] = z


_epi_call = pl.pallas_call(
    _epi_body,
    grid=(N // RE,),
    in_specs=[
        pl.BlockSpec((RE, 4), lambda i: (i, 0)),
        pl.BlockSpec((RE, 4), lambda i: (i, 0)),
        pl.BlockSpec((RE, 8), lambda i: (i, 0)),
        pl.BlockSpec((H, C), lambda i: (0, 0)),
        pl.BlockSpec((1, C), lambda i: (0, 0)),
    ],
    out_specs=[
        pl.BlockSpec((RE, H), lambda i: (i, 0)),
        pl.BlockSpec((RE, C), lambda i: (i, 0)),
    ],
    out_shape=[
        jax.ShapeDtypeStruct((N, H), jnp.float32),
        jax.ShapeDtypeStruct((N, C), jnp.float32),
    ],
)


def kernel(x, edge_index, edge_attr, W_rel, W_root, b, W_out, b_out):
    del edge_attr

    src = edge_index[0]
    dst = edge_index[1]
    pad = jnp.full((EP - E,), N, dtype=jnp.int32)
    src_p = jnp.concatenate([src, pad])
    dst_p = jnp.concatenate([dst, pad])

    x_pad = jnp.pad(x, ((0, NACC - N), (0, 0)))
    w_cat = jnp.concatenate(
        [
            W_rel[0],
            jnp.zeros((F, 1), jnp.float32),
            W_root,
            jnp.zeros((F, 1), jnp.float32),
        ],
        axis=1,
    )
    b_cat = jnp.concatenate(
        [jnp.zeros((3,), jnp.float32), jnp.ones((1,), jnp.float32), b,
         jnp.zeros((1,), jnp.float32)]
    ).reshape(1, 8)

    pre = _pre_call(x_pad, w_cat, b_cat)
    ypack = pre[:, 0:4]

    zeros = jnp.zeros((NACC, 4), jnp.float32)
    accs = _sc_call(ypack, src_p, dst_p, zeros)

    h, z = _epi_call(
        accs[:NACC], accs[NACC:], pre[:N], W_out, b_out.reshape(1, C)
    )
    return (h, z)


# speedup vs baseline: 6.0296x; 6.0296x over previous
import functools

import jax
import jax.numpy as jnp
from jax import lax
from jax.experimental import pallas as pl
from jax.experimental.pallas import tpu as pltpu
from jax.experimental.pallas import tpu_sc as plsc

N = 10000
E = 320000
F = 128
H = 3
C = 16

NC = 2
NS = 16
NW = NC * NS

K = 128
EPT = ((E + NW * K - 1) // (NW * K)) * K
EP = EPT * NW
NCHUNK = EPT // K

NACC = 10112
RT = NACC // NS
D = 8

RB = 632
RE = 1000


def _pre_body(x_ref, w_ref, b_ref, o_ref):
    o_ref[...] = (
        jnp.dot(x_ref[...], w_ref[...], preferred_element_type=jnp.float32)
        + b_ref[...]
    )


_pre_call = pl.pallas_call(
    _pre_body,
    grid=(NACC // RB,),
    in_specs=[
        pl.BlockSpec((RB, F), lambda i: (i, 0)),
        pl.BlockSpec((F, D), lambda i: (0, 0)),
        pl.BlockSpec((1, D), lambda i: (0, 0)),
    ],
    out_specs=pl.BlockSpec((RB, D), lambda i: (i, 0)),
    out_shape=jax.ShapeDtypeStruct((NACC, D), jnp.float32),
)


def _sc_body(pre_hbm, src_hbm, dst_hbm, zeros_hbm, out_hbm,
             src_v, dst_v, rows_v, acc_sh, sem):
    c = lax.axis_index("c")
    s = lax.axis_index("s")
    wid = c * NS + s

    pltpu.sync_copy(zeros_hbm, acc_sh.at[s])

    @pl.loop(0, NCHUNK)
    def _(g):
        base = wid * EPT + g * K
        pltpu.sync_copy(src_hbm.at[pl.ds(base, K)], src_v)
        pltpu.sync_copy(dst_hbm.at[pl.ds(base, K)], dst_v)
        pltpu.async_copy(pre_hbm.at[src_v], rows_v, sem).wait()
        pltpu.sync_copy(rows_v, acc_sh.at[s].at[dst_v], add=True)

    pltpu.sync_copy(acc_sh.at[s], out_hbm.at[wid])


_sc_call = functools.partial(
    pl.kernel,
    out_type=jax.ShapeDtypeStruct((NW, NACC, D), jnp.float32),
    mesh=plsc.VectorSubcoreMesh(core_axis_name="c", subcore_axis_name="s"),
    scratch_types=[
        pltpu.VMEM((K,), jnp.int32),
        pltpu.VMEM((K,), jnp.int32),
        pltpu.VMEM((K, D), jnp.float32),
        pltpu.VMEM_SHARED((NS, NACC, D), jnp.float32),
        pltpu.SemaphoreType.DMA,
    ],
    compiler_params=pltpu.CompilerParams(use_tc_tiling_on_sc=False),
)(_sc_body)


def _epi_body(acc_ref, pre_ref, w_ref, b_ref, h_ref, z_ref):
    acc = jnp.sum(acc_ref[...], axis=0)
    ssum = acc[:, 0:3]
    cnt = acc[:, 3:4]
    mean = ssum / jnp.maximum(cnt, 1.0)
    out = pre_ref[:, 4:7] + mean
    h = jnp.maximum(out, 0.0)
    z = jnp.dot(h, w_ref[...], preferred_element_type=jnp.float32) + b_ref[...]
    h_ref[...] = h
    z_ref[...] = z


_epi_call = pl.pallas_call(
    _epi_body,
    grid=(N // RE,),
    in_specs=[
        pl.BlockSpec((NW, RE, D), lambda i: (0, i, 0)),
        pl.BlockSpec((RE, D), lambda i: (i, 0)),
        pl.BlockSpec((H, C), lambda i: (0, 0)),
        pl.BlockSpec((1, C), lambda i: (0, 0)),
    ],
    out_specs=[
        pl.BlockSpec((RE, H), lambda i: (i, 0)),
        pl.BlockSpec((RE, C), lambda i: (i, 0)),
    ],
    out_shape=[
        jax.ShapeDtypeStruct((N, H), jnp.float32),
        jax.ShapeDtypeStruct((N, C), jnp.float32),
    ],
)


def kernel(x, edge_index, edge_attr, W_rel, W_root, b, W_out, b_out):
    del edge_attr

    src = edge_index[0]
    dst = edge_index[1]
    pad = jnp.full((EP - E,), N, dtype=jnp.int32)
    src_p = jnp.concatenate([src, pad])
    dst_p = jnp.concatenate([dst, pad])

    x_pad = jnp.pad(x, ((0, NACC - N), (0, 0)))
    w_cat = jnp.concatenate(
        [
            W_rel[0],
            jnp.zeros((F, 1), jnp.float32),
            W_root,
            jnp.zeros((F, 1), jnp.float32),
        ],
        axis=1,
    )
    b_cat = jnp.concatenate(
        [jnp.zeros((3,), jnp.float32), jnp.ones((1,), jnp.float32), b,
         jnp.zeros((1,), jnp.float32)]
    ).reshape(1, D)

    pre = _pre_call(x_pad, w_cat, b_cat)

    zeros = jnp.zeros((NACC, D), jnp.float32)
    accs = _sc_call(pre, src_p, dst_p, zeros)

    h, z = _epi_call(accs, pre[:N], W_out, b_out.reshape(1, C))
    return (h, z)


# speedup vs baseline: 8.0418x; 1.3337x over previous
import functools

import jax
import jax.numpy as jnp
from jax import lax
from jax.experimental import pallas as pl
from jax.experimental.pallas import tpu as pltpu
from jax.experimental.pallas import tpu_sc as plsc

N = 10000
E = 320000
F = 128
H = 3
C = 16

NC = 2
NS = 16
NW = NC * NS

K = 128
NBUF = 2
_CH = (E + NW * K - 1) // (NW * K)
NCHUNK = ((_CH + NBUF - 1) // NBUF) * NBUF
EPT = NCHUNK * K
EP = EPT * NW

NACC = 10112
RT = NACC // NS
D = 8

RB = 632
RE = 1000


def _pre_body(x_ref, w_ref, b_ref, o_ref):
    o_ref[...] = (
        jnp.dot(x_ref[...], w_ref[...], preferred_element_type=jnp.float32)
        + b_ref[...]
    )


_pre_call = pl.pallas_call(
    _pre_body,
    grid=(NACC // RB,),
    in_specs=[
        pl.BlockSpec((RB, F), lambda i: (i, 0)),
        pl.BlockSpec((F, D), lambda i: (0, 0)),
        pl.BlockSpec((1, D), lambda i: (0, 0)),
    ],
    out_specs=pl.BlockSpec((RB, D), lambda i: (i, 0)),
    out_shape=jax.ShapeDtypeStruct((NACC, D), jnp.float32),
)


def _sc_body(pre_hbm, src_hbm, dst_hbm, zeros_hbm, out_hbm,
             src_v, dst_v, rows_v, acc_sh, zsem, isem, gsems):
    c = lax.axis_index("c")
    s = lax.axis_index("s")
    wid = c * NS + s

    zcp = pltpu.async_copy(zeros_hbm, acc_sh.at[s], zsem)
    pltpu.async_copy(src_hbm.at[wid], src_v, isem).wait()
    pltpu.async_copy(dst_hbm.at[wid], dst_v, isem).wait()

    gathers = [
        pltpu.async_copy(pre_hbm.at[src_v.at[b]], rows_v.at[b], gsems[b])
        for b in range(NBUF)
    ]
    zcp.wait()

    @pl.loop(0, NCHUNK, step=NBUF)
    def _(g):
        for b in range(NBUF):
            gathers[b].wait()
            pltpu.sync_copy(rows_v.at[b], acc_sh.at[s].at[dst_v.at[g + b]],
                            add=True)

            @pl.when(g + NBUF + b < NCHUNK)
            def _():
                pltpu.async_copy(pre_hbm.at[src_v.at[g + NBUF + b]],
                                 rows_v.at[b], gsems[b])

    pltpu.sync_copy(acc_sh.at[s], out_hbm.at[wid])


_sc_call = functools.partial(
    pl.kernel,
    out_type=jax.ShapeDtypeStruct((NW, NACC, D), jnp.float32),
    mesh=plsc.VectorSubcoreMesh(core_axis_name="c", subcore_axis_name="s"),
    scratch_types=[
        pltpu.VMEM((NCHUNK, K), jnp.int32),
        pltpu.VMEM((NCHUNK, K), jnp.int32),
        pltpu.VMEM((NBUF, K, D), jnp.float32),
        pltpu.VMEM_SHARED((NS, NACC, D), jnp.float32),
        pltpu.SemaphoreType.DMA,
        pltpu.SemaphoreType.DMA,
        [pltpu.SemaphoreType.DMA] * NBUF,
    ],
    compiler_params=pltpu.CompilerParams(use_tc_tiling_on_sc=False),
)(_sc_body)


def _epi_body(acc_ref, pre_ref, w_ref, b_ref, h_ref, z_ref):
    acc = jnp.sum(acc_ref[...], axis=0)
    ssum = acc[:, 0:3]
    cnt = acc[:, 3:4]
    mean = ssum / jnp.maximum(cnt, 1.0)
    out = pre_ref[:, 4:7] + mean
    h = jnp.maximum(out, 0.0)
    z = jnp.dot(h, w_ref[...], preferred_element_type=jnp.float32) + b_ref[...]
    h_ref[...] = h
    z_ref[...] = z


_epi_call = pl.pallas_call(
    _epi_body,
    grid=(N // RE,),
    in_specs=[
        pl.BlockSpec((NW, RE, D), lambda i: (0, i, 0)),
        pl.BlockSpec((RE, D), lambda i: (i, 0)),
        pl.BlockSpec((H, C), lambda i: (0, 0)),
        pl.BlockSpec((1, C), lambda i: (0, 0)),
    ],
    out_specs=[
        pl.BlockSpec((RE, H), lambda i: (i, 0)),
        pl.BlockSpec((RE, C), lambda i: (i, 0)),
    ],
    out_shape=[
        jax.ShapeDtypeStruct((N, H), jnp.float32),
        jax.ShapeDtypeStruct((N, C), jnp.float32),
    ],
)


def kernel(x, edge_index, edge_attr, W_rel, W_root, b, W_out, b_out):
    del edge_attr

    src = edge_index[0]
    dst = edge_index[1]
    pad = jnp.full((EP - E,), N, dtype=jnp.int32)
    src_p = jnp.concatenate([src, pad]).reshape(NW, NCHUNK, K)
    dst_p = jnp.concatenate([dst, pad]).reshape(NW, NCHUNK, K)

    x_pad = jnp.pad(x, ((0, NACC - N), (0, 0)))
    w_cat = jnp.concatenate(
        [
            W_rel[0],
            jnp.zeros((F, 1), jnp.float32),
            W_root,
            jnp.zeros((F, 1), jnp.float32),
        ],
        axis=1,
    )
    b_cat = jnp.concatenate(
        [jnp.zeros((3,), jnp.float32), jnp.ones((1,), jnp.float32), b,
         jnp.zeros((1,), jnp.float32)]
    ).reshape(1, D)

    pre = _pre_call(x_pad, w_cat, b_cat)

    zeros = jnp.zeros((NACC, D), jnp.float32)
    accs = _sc_call(pre, src_p, dst_p, zeros)

    h, z = _epi_call(accs, pre[:N], W_out, b_out.reshape(1, C))
    return (h, z)


# speedup vs baseline: 14.7334x; 1.8321x over previous
import functools

import jax
import jax.numpy as jnp
from jax import lax
from jax.experimental import pallas as pl
from jax.experimental.pallas import tpu as pltpu
from jax.experimental.pallas import tpu_sc as plsc

N = 10000
E = 320000
F = 128
H = 3
C = 16

NC = 2
NS = 16
NW = NC * NS

K = 128
NBUF = 2
_CH = (E + NW * K - 1) // (NW * K)
NCHUNK = ((_CH + NBUF - 1) // NBUF) * NBUF
EPT = NCHUNK * K
EP = EPT * NW

NACC = 10240
RT = NACC // NS
D = 8
NVEC = RT * D // 16

RB = 1000
RE = 1000


def _pre_body(x_ref, w_ref, b_ref, o_ref):
    o_ref[...] = (
        jnp.dot(x_ref[...], w_ref[...], preferred_element_type=jnp.float32)
        + b_ref[...]
    )


_pre_call = pl.pallas_call(
    _pre_body,
    grid=(N // RB,),
    in_specs=[
        pl.BlockSpec((RB, F), lambda i: (i, 0)),
        pl.BlockSpec((F, D), lambda i: (0, 0)),
        pl.BlockSpec((1, D), lambda i: (0, 0)),
    ],
    out_specs=pl.BlockSpec((RB, D), lambda i: (i, 0)),
    out_shape=jax.ShapeDtypeStruct((N, D), jnp.float32),
)


def _sc_body(pre_hbm, ei_hbm, zeros_hbm, out_hbm,
             src_v, dst_v, rows_v, red_v, tmp_v, acc_sh,
             zsem, isem, gsems, rsems):
    c = lax.axis_index("c")
    s = lax.axis_index("s")
    wid = c * NS + s

    zcp = pltpu.async_copy(zeros_hbm, acc_sh.at[s], zsem)
    pltpu.async_copy(ei_hbm.at[0, wid], src_v, isem).wait()
    pltpu.async_copy(ei_hbm.at[1, wid], dst_v, isem).wait()

    gathers = [
        pltpu.async_copy(pre_hbm.at[src_v.at[b]], rows_v.at[b], gsems[b])
        for b in range(NBUF)
    ]
    zcp.wait()

    @pl.loop(0, NCHUNK, step=NBUF)
    def _(g):
        for b in range(NBUF):
            gathers[b].wait()
            pltpu.sync_copy(rows_v.at[b], acc_sh.at[s].at[dst_v.at[g + b]],
                            add=True)

            @pl.when(g + NBUF + b < NCHUNK)
            def _():
                pltpu.async_copy(pre_hbm.at[src_v.at[g + NBUF + b]],
                                 rows_v.at[b], gsems[b])

    plsc.subcore_barrier()

    iot = lax.iota(jnp.int32, 16)
    rhalf = iot >> 3
    cmask = iot & 7
    stripe = pl.ds(s * RT, RT)

    pltpu.sync_copy(acc_sh.at[0].at[stripe], red_v)

    def start(r, b):
        return pltpu.async_copy(acc_sh.at[r].at[stripe], tmp_v.at[b],
                                rsems[b])

    cps = [start(1, 0), start(2, 1)]

    def accum(b):
        cps[b].wait()

        @pl.loop(0, NVEC, unroll=4)
        def _(j):
            row = j * 2 + rhalf
            g_t = plsc.load_gather(tmp_v.at[b], [row, cmask])
            g_r = plsc.load_gather(red_v, [row, cmask])
            plsc.store_scatter(red_v, [row, cmask], g_r + g_t)

    @pl.loop(0, (NS - 2) // 2)
    def _(k):
        accum(0)
        start(2 * k + 3, 0)
        accum(1)

        @pl.when(k < (NS - 2) // 2 - 1)
        def _():
            start(2 * k + 4, 1)

    accum(0)

    pltpu.sync_copy(red_v, out_hbm.at[c].at[stripe])


_sc_call = functools.partial(
    pl.kernel,
    out_type=jax.ShapeDtypeStruct((NC, NACC, D), jnp.float32),
    mesh=plsc.VectorSubcoreMesh(core_axis_name="c", subcore_axis_name="s"),
    scratch_types=[
        pltpu.VMEM((NCHUNK, K), jnp.int32),
        pltpu.VMEM((NCHUNK, K), jnp.int32),
        pltpu.VMEM((NBUF, K, D), jnp.float32),
        pltpu.VMEM((RT, D), jnp.float32),
        pltpu.VMEM((NBUF, RT, D), jnp.float32),
        pltpu.VMEM_SHARED((NS, NACC, D), jnp.float32),
        pltpu.SemaphoreType.DMA,
        pltpu.SemaphoreType.DMA,
        [pltpu.SemaphoreType.DMA] * NBUF,
        [pltpu.SemaphoreType.DMA] * NBUF,
    ],
    compiler_params=pltpu.CompilerParams(use_tc_tiling_on_sc=False,
                                         needs_layout_passes=False),
)(_sc_body)


def _epi_body(acc_ref, pre_ref, w_ref, b_ref, h_ref, z_ref):
    acc = acc_ref[0] + acc_ref[1]
    ssum = acc[:, 0:3]
    cnt = acc[:, 3:4]
    mean = ssum / jnp.maximum(cnt, 1.0)
    out = pre_ref[:, 4:7] + mean
    h = jnp.maximum(out, 0.0)
    z = jnp.dot(h, w_ref[...], preferred_element_type=jnp.float32) + b_ref[...]
    h_ref[...] = h
    z_ref[...] = z


_epi_call = pl.pallas_call(
    _epi_body,
    grid=(N // RE,),
    in_specs=[
        pl.BlockSpec((NC, RE, D), lambda i: (0, i, 0)),
        pl.BlockSpec((RE, D), lambda i: (i, 0)),
        pl.BlockSpec((H, C), lambda i: (0, 0)),
        pl.BlockSpec((1, C), lambda i: (0, 0)),
    ],
    out_specs=[
        pl.BlockSpec((RE, H), lambda i: (i, 0)),
        pl.BlockSpec((RE, C), lambda i: (i, 0)),
    ],
    out_shape=[
        jax.ShapeDtypeStruct((N, H), jnp.float32),
        jax.ShapeDtypeStruct((N, C), jnp.float32),
    ],
)


def kernel(x, edge_index, edge_attr, W_rel, W_root, b, W_out, b_out):
    del edge_attr

    pad_blk = jnp.concatenate(
        [jnp.zeros((1, EP - E), jnp.int32), jnp.full((1, EP - E), N, jnp.int32)]
    )
    ei_p = jnp.concatenate([edge_index, pad_blk], axis=1).reshape(
        2, NW, NCHUNK, K)

    w_cat = jnp.concatenate(
        [
            W_rel[0],
            jnp.zeros((F, 1), jnp.float32),
            W_root,
            jnp.zeros((F, 1), jnp.float32),
        ],
        axis=1,
    )
    b_cat = jnp.concatenate(
        [jnp.zeros((3,), jnp.float32), jnp.ones((1,), jnp.float32), b,
         jnp.zeros((1,), jnp.float32)]
    ).reshape(1, D)

    pre = _pre_call(x, w_cat, b_cat)

    zeros = jnp.zeros((NACC, D), jnp.float32)
    accs = _sc_call(pre, ei_p, zeros)

    h, z = _epi_call(accs, pre, W_out, b_out.reshape(1, C))
    return (h, z)
